# Initial kernel scaffold; baseline (speedup 1.0000x reference)
import jax
import jax.numpy as jnp
from jax.experimental import pallas as pl


def kernel(x):
    raise NotImplementedError("write your pallas kernel here")


# speedup vs baseline: 2.1116x; 2.1116x over previous
import functools

import jax
import jax.numpy as jnp
from jax import lax
from jax.experimental import pallas as pl
from jax.experimental.pallas import tpu as pltpu
from jax.experimental.pallas import tpu_sc as plsc

LANES = 16
CSTRIP = 16


def _make_phase1(rows, cols, nw, num_cores, chunk_rows):
    rpw = rows // nw
    nchunks = rpw // chunk_rows
    ngroups = cols // LANES

    mesh = plsc.VectorSubcoreMesh(core_axis_name="c", subcore_axis_name="s")

    @functools.partial(
        pl.kernel,
        out_type=jax.ShapeDtypeStruct((nw * cols,), jnp.float32),
        mesh=mesh,
        scratch_types=[
            pltpu.VMEM((chunk_rows * cols,), jnp.float32),
            pltpu.VMEM((cols,), jnp.float32),
        ],
        compiler_params=pltpu.CompilerParams(use_tc_tiling_on_sc=False),
    )
    def phase1(x_hbm, sums_hbm, xbuf, carry):
        cid = lax.axis_index("c")
        sid = lax.axis_index("s")
        wid = sid * num_cores + cid
        base = wid * rpw * cols

        def chunk_body(i, _):
            pltpu.sync_copy(
                x_hbm.at[pl.ds(base + i * chunk_rows * cols, chunk_rows * cols)],
                xbuf,
            )
            for cg in range(ngroups // CSTRIP):
                col0 = cg * CSTRIP * LANES
                if_first = i == 0
                cs = [
                    jnp.where(
                        if_first,
                        jnp.zeros((LANES,), jnp.float32),
                        carry[pl.ds(col0 + g * LANES, LANES)],
                    )
                    for g in range(CSTRIP)
                ]

                def row_body(r, cs, _col0=col0):
                    rb = r * cols + _col0
                    return tuple(
                        cs[g] + xbuf[pl.ds(rb + g * LANES, LANES)]
                        for g in range(CSTRIP)
                    )

                cs = lax.fori_loop(0, chunk_rows, row_body, tuple(cs))
                for g in range(CSTRIP):
                    carry[pl.ds(col0 + g * LANES, LANES)] = cs[g]
            return 0

        lax.fori_loop(0, nchunks, chunk_body, 0)
        pltpu.sync_copy(carry, sums_hbm.at[pl.ds(wid * cols, cols)])

    return phase1


def _make_phase2(rows, cols, nw, num_cores, chunk_rows):
    rpw = rows // nw
    nchunks = rpw // chunk_rows
    ngroups = cols // LANES

    mesh = plsc.VectorSubcoreMesh(core_axis_name="c", subcore_axis_name="s")

    @functools.partial(
        pl.kernel,
        out_type=jax.ShapeDtypeStruct((rows * cols,), jnp.float32),
        mesh=mesh,
        scratch_types=[
            pltpu.VMEM((chunk_rows * cols,), jnp.float32),
            pltpu.VMEM((chunk_rows * cols,), jnp.float32),
            pltpu.VMEM((nw * cols,), jnp.float32),
            pltpu.VMEM((cols,), jnp.float32),
        ],
        compiler_params=pltpu.CompilerParams(use_tc_tiling_on_sc=False),
    )
    def phase2(x_hbm, sums_hbm, out_hbm, xbuf, obuf, sums_buf, carry):
        cid = lax.axis_index("c")
        sid = lax.axis_index("s")
        wid = sid * num_cores + cid
        base = wid * rpw * cols

        pltpu.sync_copy(sums_hbm, sums_buf)
        for gg in range(ngroups):
            carry[pl.ds(gg * LANES, LANES)] = jnp.zeros((LANES,), jnp.float32)

        def pref_body(v, _):
            vb = v * cols
            for gg in range(ngroups):
                off = gg * LANES
                carry[pl.ds(off, LANES)] = (
                    carry[pl.ds(off, LANES)] + sums_buf[pl.ds(vb + off, LANES)]
                )
            return 0

        lax.fori_loop(0, wid, pref_body, 0)

        def chunk_body(i, _):
            cbase = base + i * chunk_rows * cols
            pltpu.sync_copy(
                x_hbm.at[pl.ds(cbase, chunk_rows * cols)], xbuf
            )
            for cg in range(ngroups // CSTRIP):
                col0 = cg * CSTRIP * LANES
                cs = [
                    carry[pl.ds(col0 + g * LANES, LANES)] for g in range(CSTRIP)
                ]

                def row_body(r, cs, _col0=col0):
                    rb = r * cols + _col0
                    new_cs = []
                    for g in range(CSTRIP):
                        off = rb + g * LANES
                        obuf[pl.ds(off, LANES)] = cs[g]
                        new_cs.append(cs[g] + xbuf[pl.ds(off, LANES)])
                    return tuple(new_cs)

                cs = lax.fori_loop(0, chunk_rows, row_body, tuple(cs))
                for g in range(CSTRIP):
                    carry[pl.ds(col0 + g * LANES, LANES)] = cs[g]
            pltpu.sync_copy(
                obuf, out_hbm.at[pl.ds(cbase, chunk_rows * cols)]
            )
            return 0

        lax.fori_loop(0, nchunks, chunk_body, 0)

    return phase2


@jax.jit
def kernel(x):
    rows, cols = x.shape
    info = plsc.get_sparse_core_info()
    nw = info.num_cores * info.num_subcores
    p1 = _make_phase1(rows, cols, nw, info.num_cores, 64)
    p2 = _make_phase2(rows, cols, nw, info.num_cores, 32)
    x1 = x.reshape(-1)
    sums = p1(x1)
    out = p2(x1, sums)
    return out.reshape(rows, cols)


# speedup vs baseline: 2.5976x; 1.2301x over previous
import functools

import jax
import jax.numpy as jnp
from jax import lax
from jax.experimental import pallas as pl
from jax.experimental.pallas import tpu as pltpu
from jax.experimental.pallas import tpu_sc as plsc

LANES = 16
CSTRIP = 16


def _make_phase1(rows, cols, nw, num_cores, chunk_rows):
    rpw = rows // nw
    nchunks = rpw // chunk_rows
    ngroups = cols // LANES
    csz = chunk_rows * cols

    mesh = plsc.VectorSubcoreMesh(core_axis_name="c", subcore_axis_name="s")

    @functools.partial(
        pl.kernel,
        out_type=jax.ShapeDtypeStruct((nw * cols,), jnp.float32),
        mesh=mesh,
        scratch_types=[
            pltpu.VMEM((csz,), jnp.float32),
            pltpu.VMEM((csz,), jnp.float32),
            pltpu.VMEM((cols,), jnp.float32),
            pltpu.SemaphoreType.DMA,
            pltpu.SemaphoreType.DMA,
        ],
        compiler_params=pltpu.CompilerParams(use_tc_tiling_on_sc=False),
    )
    def phase1(x_hbm, sums_hbm, xbuf0, xbuf1, carry, s0, s1):
        cid = lax.axis_index("c")
        sid = lax.axis_index("s")
        wid = sid * num_cores + cid
        base = wid * rpw * cols

        def in_copy(c, buf, sem):
            return pltpu.make_async_copy(
                x_hbm.at[pl.ds(base + c * csz, csz)], buf, sem
            )

        def compute(xbuf, first):
            for cg in range(ngroups // CSTRIP):
                col0 = cg * CSTRIP * LANES
                cs = [
                    jnp.where(
                        first,
                        jnp.zeros((LANES,), jnp.float32),
                        carry[pl.ds(col0 + g * LANES, LANES)],
                    )
                    for g in range(CSTRIP)
                ]

                def row_body(r, cs, _col0=col0):
                    rb = r * cols + _col0
                    return tuple(
                        cs[g] + xbuf[pl.ds(rb + g * LANES, LANES)]
                        for g in range(CSTRIP)
                    )

                cs = lax.fori_loop(0, chunk_rows, row_body, tuple(cs))
                for g in range(CSTRIP):
                    carry[pl.ds(col0 + g * LANES, LANES)] = cs[g]

        in_copy(0, xbuf0, s0).start()

        def pair_body(i, _):
            c0 = 2 * i
            in_copy(c0 + 1, xbuf1, s1).start()
            in_copy(c0, xbuf0, s0).wait()
            compute(xbuf0, c0 == 0)

            @pl.when(c0 + 2 < nchunks)
            def _():
                in_copy(c0 + 2, xbuf0, s0).start()

            in_copy(c0 + 1, xbuf1, s1).wait()
            compute(xbuf1, False)
            return 0

        lax.fori_loop(0, nchunks // 2, pair_body, 0)
        pltpu.sync_copy(carry, sums_hbm.at[pl.ds(wid * cols, cols)])

    return phase1


def _make_phase2(rows, cols, nw, num_cores, chunk_rows):
    rpw = rows // nw
    nchunks = rpw // chunk_rows
    ngroups = cols // LANES
    csz = chunk_rows * cols

    mesh = plsc.VectorSubcoreMesh(core_axis_name="c", subcore_axis_name="s")

    @functools.partial(
        pl.kernel,
        out_type=jax.ShapeDtypeStruct((rows * cols,), jnp.float32),
        mesh=mesh,
        scratch_types=[
            pltpu.VMEM((csz,), jnp.float32),
            pltpu.VMEM((csz,), jnp.float32),
            pltpu.VMEM((csz,), jnp.float32),
            pltpu.VMEM((csz,), jnp.float32),
            pltpu.VMEM((nw * cols,), jnp.float32),
            pltpu.VMEM((cols,), jnp.float32),
            pltpu.SemaphoreType.DMA,
            pltpu.SemaphoreType.DMA,
            pltpu.SemaphoreType.DMA,
            pltpu.SemaphoreType.DMA,
        ],
        compiler_params=pltpu.CompilerParams(use_tc_tiling_on_sc=False),
    )
    def phase2(
        x_hbm, sums_hbm, out_hbm, xbuf0, xbuf1, obuf0, obuf1, sums_buf, carry,
        si0, si1, so0, so1,
    ):
        cid = lax.axis_index("c")
        sid = lax.axis_index("s")
        wid = sid * num_cores + cid
        base = wid * rpw * cols

        def in_copy(c, buf, sem):
            return pltpu.make_async_copy(
                x_hbm.at[pl.ds(base + c * csz, csz)], buf, sem
            )

        def out_copy(c, buf, sem):
            return pltpu.make_async_copy(
                buf, out_hbm.at[pl.ds(base + c * csz, csz)], sem
            )

        in_copy(0, xbuf0, si0).start()

        pltpu.sync_copy(sums_hbm, sums_buf)
        for gg in range(ngroups):
            carry[pl.ds(gg * LANES, LANES)] = jnp.zeros((LANES,), jnp.float32)

        def pref_body(v, _):
            vb = v * cols
            for gg in range(ngroups):
                off = gg * LANES
                carry[pl.ds(off, LANES)] = (
                    carry[pl.ds(off, LANES)] + sums_buf[pl.ds(vb + off, LANES)]
                )
            return 0

        lax.fori_loop(0, wid, pref_body, 0)

        def compute(xbuf, obuf):
            for cg in range(ngroups // CSTRIP):
                col0 = cg * CSTRIP * LANES
                cs = [
                    carry[pl.ds(col0 + g * LANES, LANES)] for g in range(CSTRIP)
                ]

                def row_body(r, cs, _col0=col0):
                    rb = r * cols + _col0
                    new_cs = []
                    for g in range(CSTRIP):
                        off = rb + g * LANES
                        obuf[pl.ds(off, LANES)] = cs[g]
                        new_cs.append(cs[g] + xbuf[pl.ds(off, LANES)])
                    return tuple(new_cs)

                cs = lax.fori_loop(0, chunk_rows, row_body, tuple(cs))
                for g in range(CSTRIP):
                    carry[pl.ds(col0 + g * LANES, LANES)] = cs[g]

        def pair_body(i, _):
            c0 = 2 * i
            in_copy(c0 + 1, xbuf1, si1).start()
            in_copy(c0, xbuf0, si0).wait()

            @pl.when(i > 0)
            def _():
                out_copy(c0, obuf0, so0).wait()

            compute(xbuf0, obuf0)
            out_copy(c0, obuf0, so0).start()

            @pl.when(c0 + 2 < nchunks)
            def _():
                in_copy(c0 + 2, xbuf0, si0).start()

            in_copy(c0 + 1, xbuf1, si1).wait()

            @pl.when(i > 0)
            def _():
                out_copy(c0 + 1, obuf1, so1).wait()

            compute(xbuf1, obuf1)
            out_copy(c0 + 1, obuf1, so1).start()
            return 0

        lax.fori_loop(0, nchunks // 2, pair_body, 0)
        out_copy(0, obuf0, so0).wait()
        out_copy(1, obuf1, so1).wait()

    return phase2


@jax.jit
def kernel(x):
    rows, cols = x.shape
    info = plsc.get_sparse_core_info()
    nw = info.num_cores * info.num_subcores
    p1 = _make_phase1(rows, cols, nw, info.num_cores, 32)
    p2 = _make_phase2(rows, cols, nw, info.num_cores, 16)
    x1 = x.reshape(-1)
    sums = p1(x1)
    out = p2(x1, sums)
    return out.reshape(rows, cols)


# speedup vs baseline: 5.4457x; 2.0965x over previous
import functools

import jax
import jax.numpy as jnp
from jax import lax
from jax.experimental import pallas as pl
from jax.experimental.pallas import tpu as pltpu
from jax.experimental.pallas import tpu_sc as plsc

LANES = 16
CSTRIP = 16


def _iota16():
    return lax.iota(jnp.int32, LANES)


def _make_phase1(rows, cols, nw, num_cores, chunk_rows):
    rpw = rows // nw
    nchunks = rpw // chunk_rows
    ngroups = cols // LANES

    mesh = plsc.VectorSubcoreMesh(core_axis_name="c", subcore_axis_name="s")

    @functools.partial(
        pl.kernel,
        out_type=jax.ShapeDtypeStruct((nw * cols,), jnp.float32),
        mesh=mesh,
        scratch_types=[
            pltpu.VMEM((chunk_rows, cols), jnp.float32),
            pltpu.VMEM((chunk_rows, cols), jnp.float32),
            pltpu.VMEM((cols,), jnp.float32),
            pltpu.SemaphoreType.DMA,
            pltpu.SemaphoreType.DMA,
        ],
        compiler_params=pltpu.CompilerParams(needs_layout_passes=False),
    )
    def phase1(x_hbm, sums_hbm, xbuf0, xbuf1, carry, s0, s1):
        cid = lax.axis_index("c")
        sid = lax.axis_index("s")
        wid = sid * num_cores + cid
        row_base = wid * rpw

        def in_copy(c, buf, sem):
            return pltpu.make_async_copy(
                x_hbm.at[pl.ds(row_base + c * chunk_rows, chunk_rows), :],
                buf,
                sem,
            )

        def compute(xbuf, first):
            for cg in range(ngroups // CSTRIP):
                col0 = cg * CSTRIP * LANES
                cidx = [_iota16() + (col0 + g * LANES) for g in range(CSTRIP)]
                cs = [
                    jnp.where(
                        first,
                        jnp.zeros((LANES,), jnp.float32),
                        carry[pl.ds(col0 + g * LANES, LANES)],
                    )
                    for g in range(CSTRIP)
                ]

                def row_body(r, cs, _cidx=cidx):
                    ridx = jnp.full((LANES,), r, jnp.int32)
                    return tuple(
                        cs[g] + plsc.load_gather(xbuf, [ridx, _cidx[g]])
                        for g in range(CSTRIP)
                    )

                cs = lax.fori_loop(0, chunk_rows, row_body, tuple(cs))
                for g in range(CSTRIP):
                    carry[pl.ds(col0 + g * LANES, LANES)] = cs[g]

        in_copy(0, xbuf0, s0).start()

        def pair_body(i, _):
            c0 = 2 * i
            in_copy(c0 + 1, xbuf1, s1).start()
            in_copy(c0, xbuf0, s0).wait()
            compute(xbuf0, c0 == 0)

            @pl.when(c0 + 2 < nchunks)
            def _():
                in_copy(c0 + 2, xbuf0, s0).start()

            in_copy(c0 + 1, xbuf1, s1).wait()
            compute(xbuf1, False)
            return 0

        lax.fori_loop(0, nchunks // 2, pair_body, 0)
        pltpu.sync_copy(carry, sums_hbm.at[pl.ds(wid * cols, cols)])

    return phase1


def _make_phase2(rows, cols, nw, num_cores, chunk_rows):
    rpw = rows // nw
    nchunks = rpw // chunk_rows
    ngroups = cols // LANES

    mesh = plsc.VectorSubcoreMesh(core_axis_name="c", subcore_axis_name="s")

    @functools.partial(
        pl.kernel,
        out_type=jax.ShapeDtypeStruct((rows, cols), jnp.float32),
        mesh=mesh,
        scratch_types=[
            pltpu.VMEM((chunk_rows, cols), jnp.float32),
            pltpu.VMEM((chunk_rows, cols), jnp.float32),
            pltpu.VMEM((chunk_rows, cols), jnp.float32),
            pltpu.VMEM((chunk_rows, cols), jnp.float32),
            pltpu.VMEM((nw * cols,), jnp.float32),
            pltpu.VMEM((cols,), jnp.float32),
            pltpu.SemaphoreType.DMA,
            pltpu.SemaphoreType.DMA,
            pltpu.SemaphoreType.DMA,
            pltpu.SemaphoreType.DMA,
        ],
        compiler_params=pltpu.CompilerParams(needs_layout_passes=False),
    )
    def phase2(
        x_hbm, sums_hbm, out_hbm, xbuf0, xbuf1, obuf0, obuf1, sums_buf, carry,
        si0, si1, so0, so1,
    ):
        cid = lax.axis_index("c")
        sid = lax.axis_index("s")
        wid = sid * num_cores + cid
        row_base = wid * rpw

        def in_copy(c, buf, sem):
            return pltpu.make_async_copy(
                x_hbm.at[pl.ds(row_base + c * chunk_rows, chunk_rows), :],
                buf,
                sem,
            )

        def out_copy(c, buf, sem):
            return pltpu.make_async_copy(
                buf,
                out_hbm.at[pl.ds(row_base + c * chunk_rows, chunk_rows), :],
                sem,
            )

        in_copy(0, xbuf0, si0).start()

        pltpu.sync_copy(sums_hbm, sums_buf)
        for gg in range(ngroups):
            carry[pl.ds(gg * LANES, LANES)] = jnp.zeros((LANES,), jnp.float32)

        def pref_body(v, _):
            vb = v * cols
            for gg in range(ngroups):
                off = gg * LANES
                carry[pl.ds(off, LANES)] = (
                    carry[pl.ds(off, LANES)] + sums_buf[pl.ds(vb + off, LANES)]
                )
            return 0

        lax.fori_loop(0, wid, pref_body, 0)

        def compute(xbuf, obuf):
            for cg in range(ngroups // CSTRIP):
                col0 = cg * CSTRIP * LANES
                cidx = [_iota16() + (col0 + g * LANES) for g in range(CSTRIP)]
                cs = [
                    carry[pl.ds(col0 + g * LANES, LANES)] for g in range(CSTRIP)
                ]

                def row_body(r, cs, _cidx=cidx):
                    ridx = jnp.full((LANES,), r, jnp.int32)
                    new_cs = []
                    for g in range(CSTRIP):
                        plsc.store_scatter(obuf, [ridx, _cidx[g]], cs[g])
                        new_cs.append(
                            cs[g] + plsc.load_gather(xbuf, [ridx, _cidx[g]])
                        )
                    return tuple(new_cs)

                cs = lax.fori_loop(0, chunk_rows, row_body, tuple(cs))
                for g in range(CSTRIP):
                    carry[pl.ds(col0 + g * LANES, LANES)] = cs[g]

        def pair_body(i, _):
            c0 = 2 * i
            in_copy(c0 + 1, xbuf1, si1).start()
            in_copy(c0, xbuf0, si0).wait()

            @pl.when(i > 0)
            def _():
                out_copy(c0, obuf0, so0).wait()

            compute(xbuf0, obuf0)
            out_copy(c0, obuf0, so0).start()

            @pl.when(c0 + 2 < nchunks)
            def _():
                in_copy(c0 + 2, xbuf0, si0).start()

            in_copy(c0 + 1, xbuf1, si1).wait()

            @pl.when(i > 0)
            def _():
                out_copy(c0 + 1, obuf1, so1).wait()

            compute(xbuf1, obuf1)
            out_copy(c0 + 1, obuf1, so1).start()
            return 0

        lax.fori_loop(0, nchunks // 2, pair_body, 0)
        out_copy(0, obuf0, so0).wait()
        out_copy(1, obuf1, so1).wait()

    return phase2


@jax.jit
def kernel(x):
    rows, cols = x.shape
    info = plsc.get_sparse_core_info()
    nw = info.num_cores * info.num_subcores
    p1 = _make_phase1(rows, cols, nw, info.num_cores, 32)
    p2 = _make_phase2(rows, cols, nw, info.num_cores, 16)
    sums = p1(x)
    return p2(x, sums)
